# Initial kernel scaffold; baseline (speedup 1.0000x reference)
import jax
import jax.numpy as jnp
from jax.experimental import pallas as pl


def kernel(x, table, W, b):
    raise NotImplementedError("write your pallas kernel here")


# speedup vs baseline: 8.3471x; 8.3471x over previous
import functools

import jax
import jax.numpy as jnp
from jax import lax
from jax.experimental import pallas as pl
from jax.experimental.pallas import tpu as pltpu
from jax.experimental.pallas import tpu_sc as plsc

B = 16384
L = 200
D = 16
NC, NS = 2, 16
NW = NC * NS
COLS_W = B // NW
RC = 128
NCH = COLS_W // RC
TC_ = L * RC
EPS = 1e-9


def _tv_body(t_ref, m_ref, tv_ref):
    i = pl.program_id(0)
    acc = jnp.dot(t_ref[...], m_ref[...], preferred_element_type=jnp.float32)

    @pl.when(i == 0)
    def _():
        r = lax.broadcasted_iota(jnp.int32, acc.shape, 0)
        c = lax.broadcasted_iota(jnp.int32, acc.shape, 1)
        tv_ref[...] = jnp.where((r == 0) & (c == 0), 0.0, acc)

    @pl.when(i != 0)
    def _():
        tv_ref[...] = acc


def _make_tv(table, W):
    V = table.shape[0]
    t2 = table.reshape(V // 8, 8 * D)
    m = jnp.kron(jnp.eye(8, dtype=jnp.float32), W.reshape(D, 1))
    blk = 5000
    tv2 = pl.pallas_call(
        _tv_body,
        grid=(t2.shape[0] // blk,),
        in_specs=[pl.BlockSpec((blk, 8 * D), lambda i: (i, 0)),
                  pl.BlockSpec((8 * D, 8), lambda i: (0, 0))],
        out_specs=pl.BlockSpec((blk, 8), lambda i: (i, 0)),
        out_shape=jax.ShapeDtypeStruct((t2.shape[0], 8), jnp.float32),
    )(t2, m)
    return tv2.reshape(V)


def _sc_body(xP_hbm, tv_hbm, s_hbm, idx_v, vals_v, sv, sem):
    wid = lax.axis_index("s") * NC + lax.axis_index("c")

    def chunk(ci, carry):
        row = wid * NCH + ci
        pltpu.sync_copy(xP_hbm.at[row], idx_v)
        pltpu.async_copy(tv_hbm.at[idx_v], vals_v, sem).wait()
        zero = jnp.zeros((16,), jnp.float32)

        def body(l, accs):
            return tuple(a + vals_v[pl.ds(l * RC + 16 * k, 16)]
                         for k, a in enumerate(accs))

        accs = lax.fori_loop(0, L, body, (zero,) * (RC // 16))
        for k, a in enumerate(accs):
            sv[pl.ds(16 * k, 16)] = a
        pltpu.sync_copy(sv, s_hbm.at[pl.ds(row * RC, RC)])
        return carry

    lax.fori_loop(0, NCH, chunk, 0)


_sc_gather_sum = functools.partial(
    pl.kernel,
    out_type=jax.ShapeDtypeStruct((B,), jnp.float32),
    mesh=plsc.VectorSubcoreMesh(core_axis_name="c", subcore_axis_name="s",
                                num_cores=NC, num_subcores=NS),
    scratch_types=[
        pltpu.VMEM((TC_,), jnp.int32),
        pltpu.VMEM((TC_,), jnp.float32),
        pltpu.VMEM((RC,), jnp.float32),
        pltpu.SemaphoreType.DMA,
    ],
)(_sc_body)


def _head_body(x_ref, s_ref, b_ref, o_ref):
    cnt = jnp.sum((x_ref[...] != 0).astype(jnp.float32), axis=1)
    z = s_ref[...][:, 0] / (cnt + EPS) + b_ref[0]
    p = 1.0 / (1.0 + jnp.exp(-z))
    o_ref[...] = (jnp.round(p * 10000.0) * 1e-4)[:, None]


def _head(x, s, b):
    rb = 2048
    return pl.pallas_call(
        _head_body,
        grid=(B // rb,),
        in_specs=[pl.BlockSpec((rb, L), lambda i: (i, 0)),
                  pl.BlockSpec((rb, 1), lambda i: (i, 0)),
                  pl.BlockSpec(memory_space=pltpu.SMEM)],
        out_specs=pl.BlockSpec((rb, 1), lambda i: (i, 0)),
        out_shape=jax.ShapeDtypeStruct((B, 1), jnp.float32),
    )(x, s, b)


def kernel(x, table, W, b):
    tv = _make_tv(table, W)
    xP = x.reshape(NW * NCH, RC, L).transpose(0, 2, 1).reshape(NW * NCH, TC_)
    s = _sc_gather_sum(xP, tv)
    return _head(x, s.reshape(B, 1), b)


# speedup vs baseline: 23.1473x; 2.7731x over previous
import functools

import jax
import jax.numpy as jnp
from jax import lax
from jax.experimental import pallas as pl
from jax.experimental.pallas import tpu as pltpu
from jax.experimental.pallas import tpu_sc as plsc

B = 16384
L = 200
D = 16
NC, NS = 2, 16
NW = NC * NS
NSEG = B * L // 128
SEG_W = NSEG // NW
SEGC = 160
NCH = SEG_W // SEGC
CH_ = SEGC * 128
EPS = 1e-9


TV_BLK = 8192


def _tv_body(t_ref, w_ref, tv_ref):
    tv_ref[...] = jnp.sum(t_ref[...] * w_ref[...], axis=0)


def _make_tv(table, W):
    V = table.shape[0]
    grid = -(-V // TV_BLK)
    P = grid * TV_BLK
    return pl.pallas_call(
        _tv_body,
        grid=(grid,),
        in_specs=[pl.BlockSpec((D, TV_BLK), lambda i: (0, i)),
                  pl.BlockSpec((D, 1), lambda i: (0, 0))],
        out_specs=pl.BlockSpec((TV_BLK,), lambda i: (i,)),
        out_shape=jax.ShapeDtypeStruct((P,), jnp.float32),
    )(table.T, W.reshape(D, 1))


def _sc_body(xq_hbm, tv_hbm, s_hbm,
             idx0, vals0, idx1, vals1, acc, sem0, sem1):
    wid = lax.axis_index("s") * NC + lax.axis_index("c")
    bufs = ((idx0, vals0, sem0), (idx1, vals1, sem1))

    def stage_fire(ci, buf):
        idx_b, vals_b, sem_b = buf
        off = (wid * SEG_W + ci * SEGC) * 128
        pltpu.sync_copy(xq_hbm.at[pl.ds(off, CH_)], idx_b)
        return pltpu.async_copy(tv_hbm.at[idx_b], vals_b, sem_b)

    zero = jnp.zeros((16,), jnp.float32)

    def zbody(i, carry):
        acc[pl.ds(16 * i, 16)] = zero
        return carry

    lax.fori_loop(0, B // 16, zbody, 0)

    pending = stage_fire(0, bufs[0])
    for ci in range(NCH):
        nxt = stage_fire(ci + 1, bufs[(ci + 1) % 2]) if ci + 1 < NCH else None
        pending.wait()
        vals_b = bufs[ci % 2][1]
        g0 = wid * SEG_W + ci * SEGC

        def seg_body(k, carry, vals_b=vals_b, g0=g0):
            g = g0 + k
            rbase = lax.rem(lax.div(g, 8), 128) * 128
            for m in range(8):
                a = rbase + 16 * m
                v = k * 128 + 16 * m
                acc[pl.ds(a, 16)] = acc[pl.ds(a, 16)] + vals_b[pl.ds(v, 16)]
            return carry

        lax.fori_loop(0, SEGC, seg_body, 0)
        pending = nxt
    pltpu.sync_copy(acc, s_hbm.at[wid])


_sc_gather_sum = functools.partial(
    pl.kernel,
    out_type=jax.ShapeDtypeStruct((NW, B), jnp.float32),
    mesh=plsc.VectorSubcoreMesh(core_axis_name="c", subcore_axis_name="s",
                                num_cores=NC, num_subcores=NS),
    scratch_types=[
        pltpu.VMEM((CH_,), jnp.int32),
        pltpu.VMEM((CH_,), jnp.float32),
        pltpu.VMEM((CH_,), jnp.int32),
        pltpu.VMEM((CH_,), jnp.float32),
        pltpu.VMEM((B,), jnp.float32),
        pltpu.SemaphoreType.DMA,
        pltpu.SemaphoreType.DMA,
    ],
)(_sc_body)


def _head_body(xt_ref, s_ref, b_ref, o_ref):
    cnt = jnp.sum((xt_ref[...] != 0).astype(jnp.float32), axis=0)
    s = jnp.sum(s_ref[...], axis=0)
    z = s / (cnt + EPS) + b_ref[0]
    p = 1.0 / (1.0 + jnp.exp(-z))
    o_ref[...] = (jnp.round(p * 10000.0) * 1e-4)[:, None]


def _head(xt, s, b):
    rb = 2048
    return pl.pallas_call(
        _head_body,
        grid=(B // rb,),
        in_specs=[pl.BlockSpec((L, rb), lambda i: (0, i)),
                  pl.BlockSpec((NW, rb), lambda i: (0, i)),
                  pl.BlockSpec(memory_space=pltpu.SMEM)],
        out_specs=pl.BlockSpec((rb, 1), lambda i: (i, 0)),
        out_shape=jax.ShapeDtypeStruct((B, 1), jnp.float32),
    )(xt, s, b)


def kernel(x, table, W, b):
    tv = _make_tv(table, W)
    xt = x.T
    xq = xt.reshape(L // 8, 8, B // 128, 128).transpose(0, 2, 1, 3)
    s = _sc_gather_sum(xq.reshape(B * L), tv)
    return _head(xt, s, b)


# speedup vs baseline: 31.8663x; 1.3767x over previous
import functools

import jax
import jax.numpy as jnp
from jax import lax
from jax.experimental import pallas as pl
from jax.experimental.pallas import tpu as pltpu
from jax.experimental.pallas import tpu_sc as plsc

B = 16384
L = 200
D = 16
NC, NS = 2, 16
NW = NC * NS
NSEG = B * L // 128
SEG_W = NSEG // NW
SEGC = 160
NCH = SEG_W // SEGC
CH_ = SEGC * 128
EPS = 1e-9


TV_BLK = 65536


def _tv_body(t_ref, w_ref, tv_ref):
    tv_ref[...] = jnp.dot(w_ref[...], t_ref[...],
                          preferred_element_type=jnp.float32)[0]


def _make_tv(table, W):
    V = table.shape[0]
    grid = -(-V // TV_BLK)
    P = grid * TV_BLK
    return pl.pallas_call(
        _tv_body,
        grid=(grid,),
        in_specs=[pl.BlockSpec((D, TV_BLK), lambda i: (0, i)),
                  pl.BlockSpec((1, D), lambda i: (0, 0))],
        out_specs=pl.BlockSpec((TV_BLK,), lambda i: (i,)),
        out_shape=jax.ShapeDtypeStruct((P,), jnp.float32),
    )(table.T, W)


def _sc_body(xq_hbm, tv_hbm, s_hbm,
             idx0, vals0, idx1, vals1, acc, sem0, sem1):
    wid = lax.axis_index("s") * NC + lax.axis_index("c")
    bufs = ((idx0, vals0, sem0), (idx1, vals1, sem1))

    def stage_fire(ci, buf):
        idx_b, vals_b, sem_b = buf
        off = (wid * SEG_W + ci * SEGC) * 128
        pltpu.sync_copy(xq_hbm.at[pl.ds(off, CH_)], idx_b)
        return pltpu.async_copy(tv_hbm.at[idx_b], vals_b, sem_b)

    zero = jnp.zeros((16,), jnp.float32)

    def zbody(i, carry):
        acc[pl.ds(16 * i, 16)] = zero
        return carry

    lax.fori_loop(0, B // 16, zbody, 0)

    pending = stage_fire(0, bufs[0])
    for ci in range(NCH):
        nxt = stage_fire(ci + 1, bufs[(ci + 1) % 2]) if ci + 1 < NCH else None
        pending.wait()
        vals_b = bufs[ci % 2][1]
        g0 = wid * SEG_W + ci * SEGC

        def seg_body(k, carry, vals_b=vals_b, g0=g0):
            g = g0 + k
            rbase = lax.rem(lax.div(g, 8), 128) * 128
            for m in range(8):
                a = rbase + 16 * m
                v = k * 128 + 16 * m
                acc[pl.ds(a, 16)] = acc[pl.ds(a, 16)] + vals_b[pl.ds(v, 16)]
            return carry

        lax.fori_loop(0, SEGC, seg_body, 0)
        pending = nxt
    pltpu.sync_copy(acc, s_hbm.at[wid])


_sc_gather_sum = functools.partial(
    pl.kernel,
    out_type=jax.ShapeDtypeStruct((NW, B), jnp.float32),
    mesh=plsc.VectorSubcoreMesh(core_axis_name="c", subcore_axis_name="s",
                                num_cores=NC, num_subcores=NS),
    scratch_types=[
        pltpu.VMEM((CH_,), jnp.int32),
        pltpu.VMEM((CH_,), jnp.float32),
        pltpu.VMEM((CH_,), jnp.int32),
        pltpu.VMEM((CH_,), jnp.float32),
        pltpu.VMEM((B,), jnp.float32),
        pltpu.SemaphoreType.DMA,
        pltpu.SemaphoreType.DMA,
    ],
)(_sc_body)


def _count_body(xt_ref, c_ref):
    c_ref[...] = jnp.sum((xt_ref[...] != 0).astype(jnp.float32), axis=0)


def _count(xt):
    rb = 4096
    return pl.pallas_call(
        _count_body,
        grid=(B // rb,),
        in_specs=[pl.BlockSpec((L, rb), lambda i: (0, i))],
        out_specs=pl.BlockSpec((rb,), lambda i: (i,)),
        out_shape=jax.ShapeDtypeStruct((B,), jnp.float32),
    )(xt)


def _head_body(s_ref, c_ref, b_ref, o_ref):
    s = jnp.sum(s_ref[...], axis=0)
    z = s / (c_ref[...] + EPS) + b_ref[0]
    p = 1.0 / (1.0 + jnp.exp(-z))
    o_ref[...] = (jnp.round(p * 10000.0) * 1e-4)[None, :]


def _head(s, cnt, b):
    rb = 4096
    return pl.pallas_call(
        _head_body,
        grid=(B // rb,),
        in_specs=[pl.BlockSpec((NW, rb), lambda i: (0, i)),
                  pl.BlockSpec((rb,), lambda i: (i,)),
                  pl.BlockSpec(memory_space=pltpu.SMEM)],
        out_specs=pl.BlockSpec((1, rb), lambda i: (0, i)),
        out_shape=jax.ShapeDtypeStruct((1, B), jnp.float32),
    )(s, cnt, b)


def kernel(x, table, W, b):
    tv = _make_tv(table, W)
    xt = x.T
    xq = xt.reshape(L // 8, 8, B // 128, 128).transpose(0, 2, 1, 3)
    s = _sc_gather_sum(xq.reshape(B * L), tv)
    cnt = _count(xt)
    return _head(s, cnt, b).T


# speedup vs baseline: 56.6870x; 1.7789x over previous
import functools

import jax
import jax.numpy as jnp
from jax import lax
from jax.experimental import pallas as pl
from jax.experimental.pallas import tpu as pltpu
from jax.experimental.pallas import tpu_sc as plsc

B = 16384
L = 200
D = 16
VOCAB = 1000000
NC, NS = 2, 16
NW = NC * NS
NSEG = B * L // 128
SEG_W = NSEG // NW
SEGC = 80
NCH = SEG_W // SEGC
CH_ = SEGC * 128
EPS = 1e-9


TV_BLK = 65536


def _tv_body(t_ref, w_ref, tv_ref):
    tv_ref[...] = jnp.dot(w_ref[...], t_ref[...],
                          preferred_element_type=jnp.float32)[0]


def _make_tv(table, W):
    V = table.shape[0]
    grid = -(-V // TV_BLK)
    P = 4 * 1024 * 1024
    return pl.pallas_call(
        _tv_body,
        grid=(grid,),
        in_specs=[pl.BlockSpec((D, TV_BLK), lambda i: (0, i)),
                  pl.BlockSpec((1, D), lambda i: (0, 0))],
        out_specs=pl.BlockSpec((TV_BLK,), lambda i: (i,)),
        out_shape=jax.ShapeDtypeStruct((P,), jnp.float32),
    )(table.T, W)


def _sc_body(xq_hbm, tv_hbm, s_hbm,
             idx0, vals0, idx1, vals1, acc, tvs, sem0, sem1):
    sid = lax.axis_index("s")
    wid = sid * NC + lax.axis_index("c")
    bufs = ((idx0, vals0, sem0), (idx1, vals1, sem1))

    def stage(ci, buf):
        off = (wid * SEG_W + ci * SEGC) * 128
        pltpu.sync_copy(xq_hbm.at[pl.ds(off, CH_)], buf[0])

    def fire(buf):
        return pltpu.async_copy(tvs.at[buf[0]], buf[1], buf[2])

    def stage_fire(ci, buf):
        stage(ci, buf)
        return fire(buf)

    TVC = 20000
    for j in range(-(-VOCAB // (TVC * NS))):
        c = sid + NS * j

        @pl.when(c < VOCAB // TVC)
        def _(c=c):
            pltpu.sync_copy(tv_hbm.at[pl.ds(c * TVC, TVC)],
                            vals1.at[pl.ds(0, TVC)])
            pltpu.sync_copy(vals1.at[pl.ds(0, TVC)],
                            tvs.at[pl.ds(c * TVC, TVC)])

    stage(0, bufs[0])

    zero = jnp.zeros((16,), jnp.float32)

    def zbody(i, carry):
        acc[pl.ds(16 * i, 16)] = zero
        return carry

    lax.fori_loop(0, B // 16, zbody, 0)
    plsc.subcore_barrier()
    pending = fire(bufs[0])
    for ci in range(NCH):
        nxt = stage_fire(ci + 1, bufs[(ci + 1) % 2]) if ci + 1 < NCH else None
        pending.wait()
        vals_b = bufs[ci % 2][1]
        g0 = wid * SEG_W + ci * SEGC

        def seg_body(k, carry, vals_b=vals_b, g0=g0):
            g = g0 + k
            rbase = lax.rem(lax.div(g, 8), 128) * 128
            for m in range(8):
                a = rbase + 16 * m
                v = k * 128 + 16 * m
                acc[pl.ds(a, 16)] = acc[pl.ds(a, 16)] + vals_b[pl.ds(v, 16)]
            return carry

        lax.fori_loop(0, SEGC, seg_body, 0)
        pending = nxt
    pltpu.sync_copy(acc, s_hbm.at[wid, pl.ds(0, B)])


SPAD = 65536

_sc_gather_sum = functools.partial(
    pl.kernel,
    out_type=jax.ShapeDtypeStruct((NW, SPAD), jnp.float32),
    mesh=plsc.VectorSubcoreMesh(core_axis_name="c", subcore_axis_name="s",
                                num_cores=NC, num_subcores=NS),
    scratch_types=[
        pltpu.VMEM((CH_,), jnp.int32),
        pltpu.VMEM((CH_,), jnp.float32),
        pltpu.VMEM((CH_,), jnp.int32),
        pltpu.VMEM((CH_,), jnp.float32),
        pltpu.VMEM((B,), jnp.float32),
        pltpu.VMEM_SHARED((1000000,), jnp.float32),
        pltpu.SemaphoreType.DMA,
        pltpu.SemaphoreType.DMA,
    ],
)(_sc_body)


def _count_body(xt_ref, c_ref):
    c_ref[...] = jnp.sum((xt_ref[...] != 0).astype(jnp.float32), axis=0)


def _count(xt):
    rb = 4096
    return pl.pallas_call(
        _count_body,
        grid=(B // rb,),
        in_specs=[pl.BlockSpec((L, rb), lambda i: (0, i))],
        out_specs=pl.BlockSpec((rb,), lambda i: (i,)),
        out_shape=jax.ShapeDtypeStruct((B,), jnp.float32),
    )(xt)


def _head_body(s_ref, c_ref, b_ref, o_ref):
    s = jnp.sum(s_ref[...], axis=0)
    z = s / (c_ref[...] + EPS) + b_ref[0]
    p = 1.0 / (1.0 + jnp.exp(-z))
    o_ref[...] = (jnp.round(p * 10000.0) * 1e-4)[None, :]


def _head(s, cnt, b):
    rb = 4096
    return pl.pallas_call(
        _head_body,
        grid=(B // rb,),
        in_specs=[pl.BlockSpec((NW, rb), lambda i: (0, i)),
                  pl.BlockSpec((rb,), lambda i: (i,)),
                  pl.BlockSpec(memory_space=pltpu.SMEM)],
        out_specs=pl.BlockSpec((1, rb), lambda i: (0, i)),
        out_shape=jax.ShapeDtypeStruct((1, B), jnp.float32),
    )(s, cnt, b)


def kernel(x, table, W, b):
    tv = _make_tv(table, W)
    xt = x.T
    xq = xt.reshape(L // 8, 8, B // 128, 128).transpose(0, 2, 1, 3)
    s = _sc_gather_sum(xq.reshape(B * L), tv)
    cnt = _count(xt)
    return _head(s, cnt, b).T


# speedup vs baseline: 57.0798x; 1.0069x over previous
import functools

import jax
import jax.numpy as jnp
from jax import lax
from jax.experimental import pallas as pl
from jax.experimental.pallas import tpu as pltpu
from jax.experimental.pallas import tpu_sc as plsc

B = 16384
L = 200
D = 16
VOCAB = 1000000
NC, NS = 2, 16
NW = NC * NS
NSEG = B * L // 128
SEG_W = NSEG // NW
SEGC = 100
NCH = SEG_W // SEGC
CH_ = SEGC * 128
EPS = 1e-9


TV_BLK = 65536


def _tv_body(t_ref, w_ref, tv_ref):
    tv_ref[...] = jnp.dot(w_ref[...], t_ref[...],
                          preferred_element_type=jnp.float32)[0]


def _make_tv(table, W):
    V = table.shape[0]
    grid = -(-V // TV_BLK)
    P = 4 * 1024 * 1024
    return pl.pallas_call(
        _tv_body,
        grid=(grid,),
        in_specs=[pl.BlockSpec((D, TV_BLK), lambda i: (0, i)),
                  pl.BlockSpec((1, D), lambda i: (0, 0))],
        out_specs=pl.BlockSpec((TV_BLK,), lambda i: (i,)),
        out_shape=jax.ShapeDtypeStruct((P,), jnp.float32),
    )(table.T, W)


def _sc_body(xq_hbm, tv_hbm, s_hbm,
             idx0, vals0, idx1, vals1, acc, tvs, sem0, sem1):
    sid = lax.axis_index("s")
    wid = sid * NC + lax.axis_index("c")
    bufs = ((idx0, vals0, sem0), (idx1, vals1, sem1))

    def stage(ci, buf):
        off = (wid * SEG_W + ci * SEGC) * 128
        pltpu.sync_copy(xq_hbm.at[pl.ds(off, CH_)], buf[0])

    def fire(buf):
        return pltpu.async_copy(tvs.at[buf[0]], buf[1], buf[2])

    def stage_fire(ci, buf):
        stage(ci, buf)
        return fire(buf)

    TVC = 20000
    for j in range(-(-VOCAB // (TVC * NS))):
        c = sid + NS * j

        @pl.when(c < VOCAB // TVC)
        def _(c=c):
            pltpu.sync_copy(tv_hbm.at[pl.ds(c * TVC, TVC)],
                            vals1.at[pl.ds(0, TVC)])
            pltpu.sync_copy(vals1.at[pl.ds(0, TVC)],
                            tvs.at[pl.ds(c * TVC, TVC)])

    stage(0, bufs[0])

    zero = jnp.zeros((16,), jnp.float32)

    def zbody(i, carry):
        acc[pl.ds(16 * i, 16)] = zero
        return carry

    lax.fori_loop(0, B // 16, zbody, 0)
    plsc.subcore_barrier()
    pending = fire(bufs[0])
    for ci in range(NCH):
        nxt = stage_fire(ci + 1, bufs[(ci + 1) % 2]) if ci + 1 < NCH else None
        pending.wait()
        vals_b = bufs[ci % 2][1]
        g0 = wid * SEG_W + ci * SEGC

        def seg_body(k, carry, vals_b=vals_b, g0=g0):
            g = g0 + k
            rbase = lax.rem(lax.div(g, 8), 128) * 128
            for m in range(8):
                a = rbase + 16 * m
                v = k * 128 + 16 * m
                acc[pl.ds(a, 16)] = acc[pl.ds(a, 16)] + vals_b[pl.ds(v, 16)]
            return carry

        lax.fori_loop(0, SEGC, seg_body, 0)
        pending = nxt
    pltpu.sync_copy(acc, s_hbm.at[wid, pl.ds(0, B)])


SPAD = 65536

_sc_gather_sum = functools.partial(
    pl.kernel,
    out_type=jax.ShapeDtypeStruct((NW, SPAD), jnp.float32),
    mesh=plsc.VectorSubcoreMesh(core_axis_name="c", subcore_axis_name="s",
                                num_cores=NC, num_subcores=NS),
    scratch_types=[
        pltpu.VMEM((CH_,), jnp.int32),
        pltpu.VMEM((CH_,), jnp.float32),
        pltpu.VMEM((CH_,), jnp.int32),
        pltpu.VMEM((CH_,), jnp.float32),
        pltpu.VMEM((B,), jnp.float32),
        pltpu.VMEM_SHARED((1000000,), jnp.float32),
        pltpu.SemaphoreType.DMA,
        pltpu.SemaphoreType.DMA,
    ],
)(_sc_body)


def _count_body(xt_ref, c_ref):
    c_ref[...] = jnp.sum((xt_ref[...] != 0).astype(jnp.float32), axis=0)


def _count(xt):
    rb = 4096
    return pl.pallas_call(
        _count_body,
        grid=(B // rb,),
        in_specs=[pl.BlockSpec((L, rb), lambda i: (0, i))],
        out_specs=pl.BlockSpec((rb,), lambda i: (i,)),
        out_shape=jax.ShapeDtypeStruct((B,), jnp.float32),
    )(xt)


def _head_body(s_ref, c_ref, b_ref, o_ref):
    s = jnp.sum(s_ref[...], axis=0)
    z = s / (c_ref[...] + EPS) + b_ref[0]
    p = 1.0 / (1.0 + jnp.exp(-z))
    o_ref[...] = (jnp.round(p * 10000.0) * 1e-4)[None, :]


def _head(s, cnt, b):
    rb = 4096
    return pl.pallas_call(
        _head_body,
        grid=(B // rb,),
        in_specs=[pl.BlockSpec((NW, rb), lambda i: (0, i)),
                  pl.BlockSpec((rb,), lambda i: (i,)),
                  pl.BlockSpec(memory_space=pltpu.SMEM)],
        out_specs=pl.BlockSpec((1, rb), lambda i: (0, i)),
        out_shape=jax.ShapeDtypeStruct((1, B), jnp.float32),
    )(s, cnt, b)


def kernel(x, table, W, b):
    tv = _make_tv(table, W)
    xt = x.T
    xq = xt.reshape(L // 8, 8, B // 128, 128).transpose(0, 2, 1, 3)
    s = _sc_gather_sum(xq.reshape(B * L), tv)
    cnt = _count(xt)
    return _head(s, cnt, b).T
